# baseline (speedup 1.0000x reference)
import functools

import jax
import jax.numpy as jnp
from jax import lax
from jax.experimental import pallas as pl
from jax.experimental.pallas import tpu as pltpu
from jax.experimental.pallas import tpu_sc as plsc

N = 10000
E = 320000
D = 128
G = 64

NC = 2
NS = 16
NW = NC * NS
EPT = E // NW
CH = 50
NCH = EPT // CH
NSTG = 4
PCH = NCH // NSTG
RCH = 80
NRCH = N // RCH

BN = 400
NB = N // BN

_mesh = plsc.VectorSubcoreMesh(core_axis_name="c", subcore_axis_name="s")
_sc_params = pltpu.CompilerParams(needs_layout_passes=False)


def _deg_body(dst_hbm, out_hbm, didx, deg_v):
    c = lax.axis_index("c")
    s = lax.axis_index("s")
    wid = c * NS + s

    pltpu.sync_copy(dst_hbm.at[pl.ds(wid * EPT, EPT)], didx)

    zeros16 = jnp.zeros((16,), jnp.float32)

    def _zero(i, carry):
        deg_v[pl.ds(i * 16, 16)] = zeros16
        return carry

    lax.fori_loop(0, N // 16, _zero, 0)

    ones16 = jnp.ones((16,), jnp.float32)

    def _count(i, carry):
        idx = didx[pl.ds(i * 16, 16)]
        plsc.addupdate_scatter(deg_v, [idx], ones16)
        return carry

    lax.fori_loop(0, EPT // 16, _count, 0)

    pltpu.sync_copy(deg_v, out_hbm.at[wid])


_deg = functools.partial(
    pl.kernel,
    out_type=jax.ShapeDtypeStruct((NW, N), jnp.float32),
    mesh=_mesh,
    compiler_params=_sc_params,
    scratch_types=[
        pltpu.VMEM((EPT,), jnp.int32),
        pltpu.VMEM((N,), jnp.float32),
    ],
)(_deg_body)


NBUF = 4


def _agg_body(hs_hbm, src_hbm, dst_hbm, zero_hbm, out_hbm, sidx, didx,
              rowsbuf, acc, sem0, sem1, sem2, sem3):
    c = lax.axis_index("c")
    s = lax.axis_index("s")
    wid = c * NS + s
    rows = tuple(rowsbuf.at[pl.ds(b * CH, CH)] for b in range(NBUF))
    gsems = (sem0, sem1, sem2, sem3)
    ssems = gsems
    rows0 = rows[0]

    pltpu.sync_copy(zero_hbm, rows0.at[pl.ds(0, RCH)])
    nk = (NRCH - s + NS - 1) // NS

    def _zacc(k, carry):
        off = (s + k * NS) * RCH
        pltpu.sync_copy(rows0.at[pl.ds(0, RCH)], acc.at[pl.ds(off, RCH)])
        return carry

    lax.fori_loop(0, nk, _zacc, 0)
    plsc.subcore_barrier()

    def _phase(p, carry):
        pltpu.sync_copy(src_hbm.at[wid, p], sidx)
        pltpu.sync_copy(dst_hbm.at[wid, p], didx)
        pltpu.async_copy(hs_hbm.at[sidx.at[0]], rows[0], gsems[0])
        pltpu.async_copy(hs_hbm.at[sidx.at[1]], rows[1], gsems[1])

        def _step(j, carry2):
            for b in range(NBUF):
                i = j * NBUF + b
                b2 = (b + 2) % NBUF
                pltpu.make_async_copy(hs_hbm.at[sidx.at[i]], rows[b],
                                      gsems[b]).wait()
                pltpu.async_copy(rows[b], acc.at[didx.at[i]], ssems[b],
                                 add=True)

                @pl.when(i - 2 >= 0)
                def _():
                    pltpu.make_async_copy(rows[b2], acc.at[didx.at[i - 2]],
                                          ssems[b2]).wait()

                @pl.when(i + 2 < PCH)
                def _():
                    pltpu.async_copy(hs_hbm.at[sidx.at[i + 2]], rows[b2],
                                     gsems[b2])

            return carry2

        nfull = PCH // NBUF
        lax.fori_loop(0, nfull, _step, 0)
        for i in range(nfull * NBUF, PCH):
            b = i % NBUF
            pltpu.make_async_copy(hs_hbm.at[sidx.at[i]], rows[b],
                                  gsems[b]).wait()
            pltpu.async_copy(rows[b], acc.at[didx.at[i]], ssems[b], add=True)
        for i in range(max(0, PCH - 4), PCH):
            pltpu.make_async_copy(rows[i % NBUF], acc.at[didx.at[i]],
                                  ssems[i % NBUF]).wait()
        return carry

    lax.fori_loop(0, NSTG, _phase, 0)
    plsc.subcore_barrier()

    def _wb(k, carry):
        off = (s + k * NS) * RCH
        pltpu.sync_copy(acc.at[pl.ds(off, RCH)], rows0.at[pl.ds(0, RCH)])
        pltpu.sync_copy(rows0.at[pl.ds(0, RCH)], out_hbm.at[c, pl.ds(off, RCH)])
        return carry

    lax.fori_loop(0, nk, _wb, 0)


_agg = functools.partial(
    pl.kernel,
    out_type=jax.ShapeDtypeStruct((NC, N, D), jnp.float32),
    mesh=_mesh,
    compiler_params=_sc_params,
    scratch_types=[
        pltpu.VMEM((PCH, CH), jnp.int32),
        pltpu.VMEM((PCH, CH), jnp.int32),
        pltpu.VMEM((NBUF * CH, D), jnp.float32),
        pltpu.VMEM_SHARED((N, D), jnp.float32),
        pltpu.SemaphoreType.DMA,
        pltpu.SemaphoreType.DMA,
        pltpu.SemaphoreType.DMA,
        pltpu.SemaphoreType.DMA,
    ],
)(_agg_body)


def _dis_block(deg_ref):
    deg = jnp.sum(deg_ref[...], axis=1, keepdims=True) + 1.0
    return lax.rsqrt(deg)


def _tc1_body(deg_ref, x_ref, w_ref, hs_ref):
    dis = _dis_block(deg_ref)
    h = jnp.dot(x_ref[...], w_ref[...], precision=lax.Precision.HIGHEST,
                preferred_element_type=jnp.float32)
    hs_ref[...] = h * dis


_tc1 = pl.pallas_call(
    _tc1_body,
    grid=(NB,),
    in_specs=[
        pl.BlockSpec((BN, NW), lambda i: (i, 0)),
        pl.BlockSpec((BN, D), lambda i: (i, 0)),
        pl.BlockSpec((D, D), lambda i: (0, 0)),
    ],
    out_specs=pl.BlockSpec((BN, D), lambda i: (i, 0)),
    out_shape=jax.ShapeDtypeStruct((N, D), jnp.float32),
)


def _tc2_body(acc_ref, hs_ref, deg_ref, b_ref, w_ref, out_ref):
    dis = _dis_block(deg_ref)
    t = (acc_ref[0] + acc_ref[1] + hs_ref[...]) * dis + b_ref[...]
    h1 = jnp.maximum(t, 0.0)
    out_ref[...] = jnp.dot(h1, w_ref[...], precision=lax.Precision.HIGHEST,
                           preferred_element_type=jnp.float32) * dis


_tc2 = pl.pallas_call(
    _tc2_body,
    grid=(NB,),
    in_specs=[
        pl.BlockSpec((NC, BN, D), lambda i: (0, i, 0)),
        pl.BlockSpec((BN, D), lambda i: (i, 0)),
        pl.BlockSpec((BN, NW), lambda i: (i, 0)),
        pl.BlockSpec((1, D), lambda i: (0, 0)),
        pl.BlockSpec((D, D), lambda i: (0, 0)),
    ],
    out_specs=pl.BlockSpec((BN, D), lambda i: (i, 0)),
    out_shape=jax.ShapeDtypeStruct((N, D), jnp.float32),
)


def _tc3_body(acc_ref, hs_ref, deg_ref, b_ref, batch_ref, h2_ref, pooled_ref,
              s_acc, c_acc):
    i = pl.program_id(0)
    dis = _dis_block(deg_ref)
    t = (acc_ref[0] + acc_ref[1] + hs_ref[...]) * dis + b_ref[...]
    h2 = jnp.maximum(t, 0.0)
    h2_ref[...] = h2

    bt = batch_ref[...].reshape(1, BN)
    gid = lax.broadcasted_iota(jnp.int32, (G, BN), 0)
    oh = jnp.where(gid == bt, 1.0, 0.0).astype(jnp.float32)
    sblk = jnp.dot(oh, h2, precision=lax.Precision.HIGHEST,
                   preferred_element_type=jnp.float32)
    cblk = jnp.broadcast_to(jnp.sum(oh, axis=1, keepdims=True), (G, D))

    @pl.when(i == 0)
    def _():
        s_acc[...] = sblk
        c_acc[...] = cblk

    @pl.when(i > 0)
    def _():
        s_acc[...] += sblk
        c_acc[...] += cblk

    @pl.when(i == NB - 1)
    def _():
        pooled_ref[...] = s_acc[...] / jnp.maximum(c_acc[...], 1.0)


_tc3 = pl.pallas_call(
    _tc3_body,
    grid=(NB,),
    in_specs=[
        pl.BlockSpec((NC, BN, D), lambda i: (0, i, 0)),
        pl.BlockSpec((BN, D), lambda i: (i, 0)),
        pl.BlockSpec((BN, NW), lambda i: (i, 0)),
        pl.BlockSpec((1, D), lambda i: (0, 0)),
        pl.BlockSpec((1, 1, BN), lambda i: (i, 0, 0)),
    ],
    out_specs=[
        pl.BlockSpec((BN, D), lambda i: (i, 0)),
        pl.BlockSpec((G, D), lambda i: (0, 0)),
    ],
    out_shape=[
        jax.ShapeDtypeStruct((N, D), jnp.float32),
        jax.ShapeDtypeStruct((G, D), jnp.float32),
    ],
    scratch_shapes=[
        pltpu.VMEM((G, D), jnp.float32),
        pltpu.VMEM((G, D), jnp.float32),
    ],
)


def kernel(x, edge_index, batch, W0, b0, W1, b1):
    src3 = edge_index[0].reshape(NW, NSTG, PCH, CH)
    dst3 = edge_index[1].reshape(NW, NSTG, PCH, CH)
    dst1 = edge_index[1]
    zero_rows = jnp.zeros((RCH, D), jnp.float32)

    degp = _deg(dst1)
    deg_t = degp.T

    hs0 = _tc1(deg_t, x, W0)
    acc0 = _agg(hs0, src3, dst3, zero_rows)
    hs1 = _tc2(acc0, hs0, deg_t, b0.reshape(1, D), W1)
    acc1 = _agg(hs1, src3, dst3, zero_rows)
    h2, pooled = _tc3(acc1, hs1, deg_t, b1.reshape(1, D),
                      batch.reshape(NB, 1, BN))
    return (pooled, h2)


# speedup vs baseline: 1.0967x; 1.0967x over previous
import functools

import jax
import jax.numpy as jnp
from jax import lax
from jax.experimental import pallas as pl
from jax.experimental.pallas import tpu as pltpu
from jax.experimental.pallas import tpu_sc as plsc

N = 10000
E = 320000
D = 128
G = 64

NC = 2
NS = 16
NW = NC * NS
EPT = E // NW
CH = 125
NCH = EPT // CH
NSTG = 2
PCH = NCH // NSTG
NBUF = 2
RCH = 80
NRCH = N // RCH
DCH = 128
NDCH = 79
NPAD = NDCH * DCH

BN = 400
NB = N // BN

_mesh = plsc.VectorSubcoreMesh(core_axis_name="c", subcore_axis_name="s")
_sc_params = pltpu.CompilerParams(needs_layout_passes=False)


def _deg_body(ei3, out_hbm, didx, deg_v, idv, tmp, deg_sh):
    c = lax.axis_index("c")
    s = lax.axis_index("s")
    wid = c * NS + s

    pltpu.sync_copy(ei3.at[1, wid], didx)

    zeros16 = jnp.zeros((16,), jnp.float32)
    iota16 = lax.broadcasted_iota(jnp.int32, (16,), 0)

    def _zero(i, carry):
        deg_v[pl.ds(i * 16, 16)] = zeros16
        return carry

    lax.fori_loop(0, NPAD // 16, _zero, 0)

    def _idz(i, carry):
        m = i // 8
        k = i % 8
        idv[m, pl.ds(k * 16, 16)] = iota16 + (i * 16)
        return carry

    lax.fori_loop(0, NPAD // 16, _idz, 0)

    def _zsh(k, carry):
        m = s + k * NS
        pltpu.sync_copy(deg_v.at[pl.ds(0, DCH)], deg_sh.at[pl.ds(m * DCH, DCH)])
        return carry

    nkz = (NDCH - s + NS - 1) // NS
    lax.fori_loop(0, nkz, _zsh, 0)

    ones16 = jnp.ones((16,), jnp.float32)

    def _count(i, carry):
        idx = didx[pl.ds(i * 16, 16)]
        plsc.addupdate_scatter(deg_v, [idx], ones16)
        return carry

    plsc.subcore_barrier()
    lax.fori_loop(0, EPT // 16, _count, 0)

    def _red(m, carry):
        pltpu.sync_copy(deg_v.at[pl.ds(m * DCH, DCH)], deg_sh.at[idv.at[m]],
                        add=True)
        return carry

    lax.fori_loop(0, NDCH, _red, 0)
    plsc.subcore_barrier()

    def _wb(k, carry):
        c2 = s + k * NS
        pltpu.sync_copy(deg_sh.at[pl.ds(c2 * RCH, RCH)], tmp)
        pltpu.sync_copy(tmp, out_hbm.at[c, c2])
        return carry

    nk = (NRCH - s + NS - 1) // NS
    lax.fori_loop(0, nk, _wb, 0)


_deg = functools.partial(
    pl.kernel,
    out_type=jax.ShapeDtypeStruct((NC, NRCH, RCH), jnp.float32),
    mesh=_mesh,
    compiler_params=_sc_params,
    scratch_types=[
        pltpu.VMEM((EPT,), jnp.int32),
        pltpu.VMEM((NPAD,), jnp.float32),
        pltpu.VMEM((NDCH, DCH), jnp.int32),
        pltpu.VMEM((RCH,), jnp.float32),
        pltpu.VMEM_SHARED((NPAD,), jnp.float32),
    ],
)(_deg_body)


def _agg_body(hs_hbm, ei5, zero_hbm, out_hbm, sidx, didx, rowsbuf, acc,
              sem0, sem1):
    c = lax.axis_index("c")
    s = lax.axis_index("s")
    wid = c * NS + s
    rows = tuple(rowsbuf.at[pl.ds(b * CH, CH)] for b in range(NBUF))
    sems = (sem0, sem1)
    rows0 = rows[0]

    pltpu.sync_copy(zero_hbm, rows0.at[pl.ds(0, RCH)])
    nk = (NRCH - s + NS - 1) // NS

    def _zacc(k, carry):
        off = (s + k * NS) * RCH
        pltpu.sync_copy(rows0.at[pl.ds(0, RCH)], acc.at[pl.ds(off, RCH)])
        return carry

    lax.fori_loop(0, nk, _zacc, 0)
    plsc.subcore_barrier()

    def _phase(p, carry):
        pltpu.sync_copy(ei5.at[0, wid, p], sidx)
        pltpu.sync_copy(ei5.at[1, wid, p], didx)
        for b in range(NBUF):
            pltpu.async_copy(hs_hbm.at[sidx.at[b]], rows[b], sems[b])

        def _step(j, carry2):
            for b in range(NBUF):
                i = j * NBUF + b
                pltpu.make_async_copy(hs_hbm.at[sidx.at[i]], rows[b],
                                      sems[b]).wait()
                pltpu.sync_copy(rows[b], acc.at[didx.at[i]], add=True)

                @pl.when(j < PCH // NBUF - 1)
                def _():
                    pltpu.async_copy(hs_hbm.at[sidx.at[i + NBUF]], rows[b],
                                     sems[b])

            return carry2

        lax.fori_loop(0, PCH // NBUF, _step, 0)
        return carry

    lax.fori_loop(0, NSTG, _phase, 0)
    plsc.subcore_barrier()

    def _wb(k, carry):
        off = (s + k * NS) * RCH
        pltpu.sync_copy(acc.at[pl.ds(off, RCH)], rows0.at[pl.ds(0, RCH)])
        pltpu.sync_copy(rows0.at[pl.ds(0, RCH)], out_hbm.at[c, pl.ds(off, RCH)])
        return carry

    lax.fori_loop(0, nk, _wb, 0)


_agg = functools.partial(
    pl.kernel,
    out_type=jax.ShapeDtypeStruct((NC, N, D), jnp.float32),
    mesh=_mesh,
    compiler_params=_sc_params,
    scratch_types=[
        pltpu.VMEM((PCH, CH), jnp.int32),
        pltpu.VMEM((PCH, CH), jnp.int32),
        pltpu.VMEM((NBUF * CH, D), jnp.float32),
        pltpu.VMEM_SHARED((N, D), jnp.float32),
        pltpu.SemaphoreType.DMA,
        pltpu.SemaphoreType.DMA,
    ],
)(_agg_body)


def _dis_block(deg_ref):
    deg = deg_ref[0] + deg_ref[1] + 1.0
    return lax.rsqrt(deg)


def _tc0_body(x_ref, w_ref, h_ref):
    h_ref[...] = jnp.dot(x_ref[...], w_ref[...],
                         precision=lax.Precision.HIGHEST,
                         preferred_element_type=jnp.float32)


_tc0 = pl.pallas_call(
    _tc0_body,
    grid=(NB,),
    in_specs=[
        pl.BlockSpec((BN, D), lambda i: (i, 0)),
        pl.BlockSpec((D, D), lambda i: (0, 0)),
    ],
    out_specs=pl.BlockSpec((BN, D), lambda i: (i, 0)),
    out_shape=jax.ShapeDtypeStruct((N, D), jnp.float32),
)


def _tc1_body(deg_ref, h_ref, hs_ref):
    hs_ref[...] = h_ref[...] * _dis_block(deg_ref)


_tc1 = pl.pallas_call(
    _tc1_body,
    grid=(NB,),
    in_specs=[
        pl.BlockSpec((NC, BN, 1), lambda i: (0, i, 0)),
        pl.BlockSpec((BN, D), lambda i: (i, 0)),
    ],
    out_specs=pl.BlockSpec((BN, D), lambda i: (i, 0)),
    out_shape=jax.ShapeDtypeStruct((N, D), jnp.float32),
)


def _tc2_body(acc_ref, hs_ref, deg_ref, b_ref, w_ref, out_ref):
    dis = _dis_block(deg_ref)
    t = (acc_ref[0] + acc_ref[1] + hs_ref[...]) * dis + b_ref[...]
    h1 = jnp.maximum(t, 0.0)
    out_ref[...] = jnp.dot(h1, w_ref[...], precision=lax.Precision.HIGHEST,
                           preferred_element_type=jnp.float32) * dis


_tc2 = pl.pallas_call(
    _tc2_body,
    grid=(NB,),
    in_specs=[
        pl.BlockSpec((NC, BN, D), lambda i: (0, i, 0)),
        pl.BlockSpec((BN, D), lambda i: (i, 0)),
        pl.BlockSpec((NC, BN, 1), lambda i: (0, i, 0)),
        pl.BlockSpec((1, D), lambda i: (0, 0)),
        pl.BlockSpec((D, D), lambda i: (0, 0)),
    ],
    out_specs=pl.BlockSpec((BN, D), lambda i: (i, 0)),
    out_shape=jax.ShapeDtypeStruct((N, D), jnp.float32),
)


def _tc3_body(acc_ref, hs_ref, deg_ref, b_ref, batch_ref, h2_ref, pooled_ref,
              s_acc, c_acc):
    i = pl.program_id(0)
    dis = _dis_block(deg_ref)
    t = (acc_ref[0] + acc_ref[1] + hs_ref[...]) * dis + b_ref[...]
    h2 = jnp.maximum(t, 0.0)
    h2_ref[...] = h2

    bt = batch_ref[...].reshape(1, BN)
    gid = lax.broadcasted_iota(jnp.int32, (G, BN), 0)
    oh = jnp.where(gid == bt, 1.0, 0.0).astype(jnp.float32)
    sblk = jnp.dot(oh, h2, precision=lax.Precision.HIGHEST,
                   preferred_element_type=jnp.float32)
    cblk = jnp.broadcast_to(jnp.sum(oh, axis=1, keepdims=True), (G, D))

    @pl.when(i == 0)
    def _():
        s_acc[...] = sblk
        c_acc[...] = cblk

    @pl.when(i > 0)
    def _():
        s_acc[...] += sblk
        c_acc[...] += cblk

    @pl.when(i == NB - 1)
    def _():
        pooled_ref[...] = s_acc[...] / jnp.maximum(c_acc[...], 1.0)


_tc3 = pl.pallas_call(
    _tc3_body,
    grid=(NB,),
    in_specs=[
        pl.BlockSpec((NC, BN, D), lambda i: (0, i, 0)),
        pl.BlockSpec((BN, D), lambda i: (i, 0)),
        pl.BlockSpec((NC, BN, 1), lambda i: (0, i, 0)),
        pl.BlockSpec((1, D), lambda i: (0, 0)),
        pl.BlockSpec((1, 1, BN), lambda i: (i, 0, 0)),
    ],
    out_specs=[
        pl.BlockSpec((BN, D), lambda i: (i, 0)),
        pl.BlockSpec((G, D), lambda i: (0, 0)),
    ],
    out_shape=[
        jax.ShapeDtypeStruct((N, D), jnp.float32),
        jax.ShapeDtypeStruct((G, D), jnp.float32),
    ],
    scratch_shapes=[
        pltpu.VMEM((G, D), jnp.float32),
        pltpu.VMEM((G, D), jnp.float32),
    ],
)


def kernel(x, edge_index, batch, W0, b0, W1, b1):
    ei3 = edge_index.reshape(2, NW, EPT)
    ei5 = edge_index.reshape(2, NW, NSTG, PCH, CH)
    zero_rows = jnp.zeros((RCH, D), jnp.float32)

    h0 = _tc0(x, W0)
    degp = _deg(ei3)
    deg2 = degp.reshape(NC, N, 1)

    hs0 = _tc1(deg2, h0)
    acc0 = _agg(hs0, ei5, zero_rows)
    hs1 = _tc2(acc0, hs0, deg2, b0.reshape(1, D), W1)
    acc1 = _agg(hs1, ei5, zero_rows)
    h2, pooled = _tc3(acc1, hs1, deg2, b1.reshape(1, D),
                      batch.reshape(NB, 1, BN))
    return (pooled, h2)


# speedup vs baseline: 1.2988x; 1.1843x over previous
import functools

import jax
import jax.numpy as jnp
from jax import lax
from jax.experimental import pallas as pl
from jax.experimental.pallas import tpu as pltpu
from jax.experimental.pallas import tpu_sc as plsc

N = 10000
E = 320000
D = 128
G = 64

NC = 2
NS = 16
NW = NC * NS
EPT = E // NW
CH = 125
NCH = EPT // CH
NSTG = 2
PCH = NCH // NSTG
NBUF = 2
RCH = 80
NRCH = N // RCH
DCH = 128
NDCH = 79
NPAD = NDCH * DCH

BN = 2000
NB = N // BN

_mesh = plsc.VectorSubcoreMesh(core_axis_name="c", subcore_axis_name="s")
_sc_params = pltpu.CompilerParams(needs_layout_passes=False)


def _deg_body(ei5, out_hbm, didx5, deg_v):
    c = lax.axis_index("c")
    s = lax.axis_index("s")
    wid = c * NS + s

    pltpu.sync_copy(ei5.at[1, wid, 0], didx5.at[pl.ds(0, PCH)])
    pltpu.sync_copy(ei5.at[1, wid, 1], didx5.at[pl.ds(PCH, PCH)])

    zeros16 = jnp.zeros((16,), jnp.float32)

    def _zero(i, carry):
        deg_v[pl.ds(i * 16, 16)] = zeros16
        return carry

    lax.fori_loop(0, N // 16, _zero, 0)

    ones16 = jnp.ones((16,), jnp.float32)
    iota16 = lax.broadcasted_iota(jnp.int32, (16,), 0)

    def _count(t, carry):
        e = t * 16 + iota16
        idx = plsc.load_gather(didx5, [e // CH, e % CH])
        plsc.addupdate_scatter(deg_v, [idx], ones16)
        return carry

    lax.fori_loop(0, EPT // 16, _count, 0)

    pltpu.sync_copy(deg_v, out_hbm.at[wid])


_deg = functools.partial(
    pl.kernel,
    out_type=jax.ShapeDtypeStruct((NW, N), jnp.float32),
    mesh=_mesh,
    compiler_params=_sc_params,
    scratch_types=[
        pltpu.VMEM((NCH, CH), jnp.int32),
        pltpu.VMEM((N,), jnp.float32),
    ],
)(_deg_body)


def _agg_body(hs_hbm, ei5, zero_hbm, out_hbm, sidx, didx, rowsbuf, acc,
              sem0, sem1):
    c = lax.axis_index("c")
    s = lax.axis_index("s")
    wid = c * NS + s
    rows = tuple(rowsbuf.at[pl.ds(b * CH, CH)] for b in range(NBUF))
    sems = (sem0, sem1)
    rows0 = rows[0]

    pltpu.sync_copy(zero_hbm, rows0.at[pl.ds(0, RCH)])
    nk = (NRCH - s + NS - 1) // NS

    def _zacc(k, carry):
        off = (s + k * NS) * RCH
        pltpu.sync_copy(rows0.at[pl.ds(0, RCH)], acc.at[pl.ds(off, RCH)])
        return carry

    lax.fori_loop(0, nk, _zacc, 0)
    plsc.subcore_barrier()

    def _phase(p, carry):
        pltpu.sync_copy(ei5.at[0, wid, p], sidx)
        pltpu.sync_copy(ei5.at[1, wid, p], didx)
        for b in range(NBUF):
            pltpu.async_copy(hs_hbm.at[sidx.at[b]], rows[b], sems[b])

        def _step(j, carry2):
            for b in range(NBUF):
                i = j * NBUF + b
                pltpu.make_async_copy(hs_hbm.at[sidx.at[i]], rows[b],
                                      sems[b]).wait()
                pltpu.sync_copy(rows[b], acc.at[didx.at[i]], add=True)

                @pl.when(j < PCH // NBUF - 1)
                def _():
                    pltpu.async_copy(hs_hbm.at[sidx.at[i + NBUF]], rows[b],
                                     sems[b])

            return carry2

        lax.fori_loop(0, PCH // NBUF, _step, 0)
        return carry

    lax.fori_loop(0, NSTG, _phase, 0)
    plsc.subcore_barrier()

    def _wb(k, carry):
        off = (s + k * NS) * RCH
        pltpu.sync_copy(acc.at[pl.ds(off, RCH)], rows0.at[pl.ds(0, RCH)])
        pltpu.sync_copy(rows0.at[pl.ds(0, RCH)], out_hbm.at[c, pl.ds(off, RCH)])
        return carry

    lax.fori_loop(0, nk, _wb, 0)


_agg = functools.partial(
    pl.kernel,
    out_type=jax.ShapeDtypeStruct((NC, N, D), jnp.float32),
    mesh=_mesh,
    compiler_params=_sc_params,
    scratch_types=[
        pltpu.VMEM((PCH, CH), jnp.int32),
        pltpu.VMEM((PCH, CH), jnp.int32),
        pltpu.VMEM((NBUF * CH, D), jnp.float32),
        pltpu.VMEM_SHARED((N, D), jnp.float32),
        pltpu.SemaphoreType.DMA,
        pltpu.SemaphoreType.DMA,
    ],
)(_agg_body)


def _dis_block(deg_ref):
    deg = jnp.sum(deg_ref[...], axis=1, keepdims=True) + 1.0
    return lax.rsqrt(deg)


def _tc0_body(x_ref, w_ref, h_ref):
    h_ref[...] = jnp.dot(x_ref[...], w_ref[...],
                         precision=lax.Precision.HIGHEST,
                         preferred_element_type=jnp.float32)


_tc0 = pl.pallas_call(
    _tc0_body,
    grid=(NB,),
    in_specs=[
        pl.BlockSpec((BN, D), lambda i: (i, 0)),
        pl.BlockSpec((D, D), lambda i: (0, 0)),
    ],
    out_specs=pl.BlockSpec((BN, D), lambda i: (i, 0)),
    out_shape=jax.ShapeDtypeStruct((N, D), jnp.float32),
)


def _tc1_body(deg_ref, h_ref, hs_ref):
    hs_ref[...] = h_ref[...] * _dis_block(deg_ref)


_tc1 = pl.pallas_call(
    _tc1_body,
    grid=(NB,),
    in_specs=[
        pl.BlockSpec((BN, NW), lambda i: (i, 0)),
        pl.BlockSpec((BN, D), lambda i: (i, 0)),
    ],
    out_specs=pl.BlockSpec((BN, D), lambda i: (i, 0)),
    out_shape=jax.ShapeDtypeStruct((N, D), jnp.float32),
)


def _tc2_body(acc_ref, hs_ref, deg_ref, b_ref, w_ref, out_ref):
    dis = _dis_block(deg_ref)
    t = (acc_ref[0] + acc_ref[1] + hs_ref[...]) * dis + b_ref[...]
    h1 = jnp.maximum(t, 0.0)
    out_ref[...] = jnp.dot(h1, w_ref[...], precision=lax.Precision.HIGHEST,
                           preferred_element_type=jnp.float32) * dis


_tc2 = pl.pallas_call(
    _tc2_body,
    grid=(NB,),
    in_specs=[
        pl.BlockSpec((NC, BN, D), lambda i: (0, i, 0)),
        pl.BlockSpec((BN, D), lambda i: (i, 0)),
        pl.BlockSpec((BN, NW), lambda i: (i, 0)),
        pl.BlockSpec((1, D), lambda i: (0, 0)),
        pl.BlockSpec((D, D), lambda i: (0, 0)),
    ],
    out_specs=pl.BlockSpec((BN, D), lambda i: (i, 0)),
    out_shape=jax.ShapeDtypeStruct((N, D), jnp.float32),
)


def _tc3_body(acc_ref, hs_ref, deg_ref, b_ref, batch_ref, h2_ref, pooled_ref,
              s_acc, c_acc):
    i = pl.program_id(0)
    dis = _dis_block(deg_ref)
    t = (acc_ref[0] + acc_ref[1] + hs_ref[...]) * dis + b_ref[...]
    h2 = jnp.maximum(t, 0.0)
    h2_ref[...] = h2

    bt = batch_ref[...].reshape(1, BN)
    gid = lax.broadcasted_iota(jnp.int32, (G, BN), 0)
    oh = jnp.where(gid == bt, 1.0, 0.0).astype(jnp.float32)
    sblk = jnp.dot(oh, h2, precision=lax.Precision.HIGHEST,
                   preferred_element_type=jnp.float32)
    cblk = jnp.broadcast_to(jnp.sum(oh, axis=1, keepdims=True), (G, D))

    @pl.when(i == 0)
    def _():
        s_acc[...] = sblk
        c_acc[...] = cblk

    @pl.when(i > 0)
    def _():
        s_acc[...] += sblk
        c_acc[...] += cblk

    @pl.when(i == NB - 1)
    def _():
        pooled_ref[...] = s_acc[...] / jnp.maximum(c_acc[...], 1.0)


_tc3 = pl.pallas_call(
    _tc3_body,
    grid=(NB,),
    in_specs=[
        pl.BlockSpec((NC, BN, D), lambda i: (0, i, 0)),
        pl.BlockSpec((BN, D), lambda i: (i, 0)),
        pl.BlockSpec((BN, NW), lambda i: (i, 0)),
        pl.BlockSpec((1, D), lambda i: (0, 0)),
        pl.BlockSpec((1, 1, BN), lambda i: (i, 0, 0)),
    ],
    out_specs=[
        pl.BlockSpec((BN, D), lambda i: (i, 0)),
        pl.BlockSpec((G, D), lambda i: (0, 0)),
    ],
    out_shape=[
        jax.ShapeDtypeStruct((N, D), jnp.float32),
        jax.ShapeDtypeStruct((G, D), jnp.float32),
    ],
    scratch_shapes=[
        pltpu.VMEM((G, D), jnp.float32),
        pltpu.VMEM((G, D), jnp.float32),
    ],
)


def kernel(x, edge_index, batch, W0, b0, W1, b1):
    ei5 = edge_index.reshape(2, NW, NSTG, PCH, CH)
    zero_rows = jnp.zeros((RCH, D), jnp.float32)

    h0 = _tc0(x, W0)
    degp = _deg(ei5)
    deg_t = degp.T

    hs0 = _tc1(deg_t, h0)
    acc0 = _agg(hs0, ei5, zero_rows)
    hs1 = _tc2(acc0, hs0, deg_t, b0.reshape(1, D), W1)
    acc1 = _agg(hs1, ei5, zero_rows)
    h2, pooled = _tc3(acc1, hs1, deg_t, b1.reshape(1, D),
                      batch.reshape(NB, 1, BN))
    return (pooled, h2)


# speedup vs baseline: 1.3067x; 1.0061x over previous
import functools

import jax
import jax.numpy as jnp
from jax import lax
from jax.experimental import pallas as pl
from jax.experimental.pallas import tpu as pltpu
from jax.experimental.pallas import tpu_sc as plsc

N = 10000
E = 320000
D = 128
G = 64

NC = 2
NS = 16
NW = NC * NS
EPT = E // NW
CH = 125
NCH = EPT // CH
NSTG = 4
PCH = NCH // NSTG
NBUF = 2
RCH = 80
NRCH = N // RCH
DCH = 128
NDCH = 79
NPAD = NDCH * DCH

BN = 2000
NB = N // BN

_mesh = plsc.VectorSubcoreMesh(core_axis_name="c", subcore_axis_name="s")
_sc_params = pltpu.CompilerParams(needs_layout_passes=False)


def _deg_body(ei5, out_hbm, didx5, deg_v):
    c = lax.axis_index("c")
    s = lax.axis_index("s")
    wid = c * NS + s

    for p in range(NSTG):
        pltpu.sync_copy(ei5.at[1, wid, p], didx5.at[pl.ds(p * PCH, PCH)])

    zeros16 = jnp.zeros((16,), jnp.float32)

    def _zero(i, carry):
        deg_v[pl.ds(i * 16, 16)] = zeros16
        return carry

    lax.fori_loop(0, N // 16, _zero, 0)

    ones16 = jnp.ones((16,), jnp.float32)
    iota16 = lax.broadcasted_iota(jnp.int32, (16,), 0)

    def _count(t, carry):
        e = t * 16 + iota16
        idx = plsc.load_gather(didx5, [e // CH, e % CH])
        plsc.addupdate_scatter(deg_v, [idx], ones16)
        return carry

    lax.fori_loop(0, EPT // 16, _count, 0)

    pltpu.sync_copy(deg_v, out_hbm.at[wid])


_deg = functools.partial(
    pl.kernel,
    out_type=jax.ShapeDtypeStruct((NW, N), jnp.float32),
    mesh=_mesh,
    compiler_params=_sc_params,
    scratch_types=[
        pltpu.VMEM((NCH, CH), jnp.int32),
        pltpu.VMEM((N,), jnp.float32),
    ],
)(_deg_body)


def _agg_body(hs_hbm, ei5, zero_hbm, out_hbm, sidx, didx, rowsbuf, acc,
              sem0, sem1, semi):
    c = lax.axis_index("c")
    s = lax.axis_index("s")
    wid = c * NS + s
    rows = tuple(rowsbuf.at[pl.ds(b * CH, CH)] for b in range(NBUF))
    sems = (sem0, sem1)
    rows0 = rows[0]

    pltpu.async_copy(ei5.at[0, wid, 0], sidx.at[0], semi)
    pltpu.async_copy(ei5.at[1, wid, 0], didx.at[0], semi)

    pltpu.sync_copy(zero_hbm, rows0.at[pl.ds(0, RCH)])
    nk = (NRCH - s + NS - 1) // NS

    def _zacc(k, carry):
        off = (s + k * NS) * RCH
        pltpu.sync_copy(rows0.at[pl.ds(0, RCH)], acc.at[pl.ds(off, RCH)])
        return carry

    lax.fori_loop(0, nk, _zacc, 0)
    pltpu.make_async_copy(ei5.at[0, wid, 0], sidx.at[0], semi).wait()
    pltpu.make_async_copy(ei5.at[1, wid, 0], didx.at[0], semi).wait()
    plsc.subcore_barrier()

    for p in range(NSTG):
        pb = p % 2
        sidx_p = sidx.at[pb]
        didx_p = didx.at[pb]
        if p < NSTG - 1:
            pltpu.async_copy(ei5.at[0, wid, p + 1], sidx.at[1 - pb], semi)
            pltpu.async_copy(ei5.at[1, wid, p + 1], didx.at[1 - pb], semi)
        for b in range(NBUF):
            pltpu.async_copy(hs_hbm.at[sidx_p.at[b]], rows[b], sems[b])

        def _step(j, carry2, sidx_p=sidx_p, didx_p=didx_p):
            for b in range(NBUF):
                i = j * NBUF + b
                pltpu.make_async_copy(hs_hbm.at[sidx_p.at[i]], rows[b],
                                      sems[b]).wait()
                pltpu.sync_copy(rows[b], acc.at[didx_p.at[i]], add=True)

                @pl.when(j < PCH // NBUF - 1)
                def _():
                    pltpu.async_copy(hs_hbm.at[sidx_p.at[i + NBUF]], rows[b],
                                     sems[b])

            return carry2

        lax.fori_loop(0, PCH // NBUF, _step, 0)
        if p < NSTG - 1:
            pltpu.make_async_copy(ei5.at[0, wid, p + 1], sidx.at[1 - pb],
                                  semi).wait()
            pltpu.make_async_copy(ei5.at[1, wid, p + 1], didx.at[1 - pb],
                                  semi).wait()
    plsc.subcore_barrier()

    def _wb(k, carry):
        off = (s + k * NS) * RCH
        pltpu.sync_copy(acc.at[pl.ds(off, RCH)], out_hbm.at[c, pl.ds(off, RCH)])
        return carry

    lax.fori_loop(0, nk, _wb, 0)


_agg = functools.partial(
    pl.kernel,
    out_type=jax.ShapeDtypeStruct((NC, N, D), jnp.float32),
    mesh=_mesh,
    compiler_params=_sc_params,
    scratch_types=[
        pltpu.VMEM((2, PCH, CH), jnp.int32),
        pltpu.VMEM((2, PCH, CH), jnp.int32),
        pltpu.VMEM((NBUF * CH, D), jnp.float32),
        pltpu.VMEM_SHARED((N, D), jnp.float32),
        pltpu.SemaphoreType.DMA,
        pltpu.SemaphoreType.DMA,
        pltpu.SemaphoreType.DMA,
    ],
)(_agg_body)


def _dis_block(deg_ref):
    deg = jnp.sum(deg_ref[...], axis=1, keepdims=True) + 1.0
    return lax.rsqrt(deg)


def _tc0_body(x_ref, w_ref, h_ref):
    h_ref[...] = jnp.dot(x_ref[...], w_ref[...],
                         precision=lax.Precision.HIGHEST,
                         preferred_element_type=jnp.float32)


_tc0 = pl.pallas_call(
    _tc0_body,
    grid=(NB,),
    in_specs=[
        pl.BlockSpec((BN, D), lambda i: (i, 0)),
        pl.BlockSpec((D, D), lambda i: (0, 0)),
    ],
    out_specs=pl.BlockSpec((BN, D), lambda i: (i, 0)),
    out_shape=jax.ShapeDtypeStruct((N, D), jnp.float32),
)


def _tc1_body(deg_ref, h_ref, hs_ref):
    hs_ref[...] = h_ref[...] * _dis_block(deg_ref)


_tc1 = pl.pallas_call(
    _tc1_body,
    grid=(NB,),
    in_specs=[
        pl.BlockSpec((BN, NW), lambda i: (i, 0)),
        pl.BlockSpec((BN, D), lambda i: (i, 0)),
    ],
    out_specs=pl.BlockSpec((BN, D), lambda i: (i, 0)),
    out_shape=jax.ShapeDtypeStruct((N, D), jnp.float32),
)


def _tc2_body(acc_ref, hs_ref, deg_ref, b_ref, w_ref, out_ref):
    dis = _dis_block(deg_ref)
    t = (acc_ref[0] + acc_ref[1] + hs_ref[...]) * dis + b_ref[...]
    h1 = jnp.maximum(t, 0.0)
    out_ref[...] = jnp.dot(h1, w_ref[...], precision=lax.Precision.HIGHEST,
                           preferred_element_type=jnp.float32) * dis


_tc2 = pl.pallas_call(
    _tc2_body,
    grid=(NB,),
    in_specs=[
        pl.BlockSpec((NC, BN, D), lambda i: (0, i, 0)),
        pl.BlockSpec((BN, D), lambda i: (i, 0)),
        pl.BlockSpec((BN, NW), lambda i: (i, 0)),
        pl.BlockSpec((1, D), lambda i: (0, 0)),
        pl.BlockSpec((D, D), lambda i: (0, 0)),
    ],
    out_specs=pl.BlockSpec((BN, D), lambda i: (i, 0)),
    out_shape=jax.ShapeDtypeStruct((N, D), jnp.float32),
)


def _tc3_body(acc_ref, hs_ref, deg_ref, b_ref, batch_ref, h2_ref, pooled_ref,
              s_acc, c_acc):
    i = pl.program_id(0)
    dis = _dis_block(deg_ref)
    t = (acc_ref[0] + acc_ref[1] + hs_ref[...]) * dis + b_ref[...]
    h2 = jnp.maximum(t, 0.0)
    h2_ref[...] = h2

    bt = batch_ref[...].reshape(1, BN)
    gid = lax.broadcasted_iota(jnp.int32, (G, BN), 0)
    oh = jnp.where(gid == bt, 1.0, 0.0).astype(jnp.float32)
    sblk = jnp.dot(oh, h2, precision=lax.Precision.HIGHEST,
                   preferred_element_type=jnp.float32)
    cblk = jnp.broadcast_to(jnp.sum(oh, axis=1, keepdims=True), (G, D))

    @pl.when(i == 0)
    def _():
        s_acc[...] = sblk
        c_acc[...] = cblk

    @pl.when(i > 0)
    def _():
        s_acc[...] += sblk
        c_acc[...] += cblk

    @pl.when(i == NB - 1)
    def _():
        pooled_ref[...] = s_acc[...] / jnp.maximum(c_acc[...], 1.0)


_tc3 = pl.pallas_call(
    _tc3_body,
    grid=(NB,),
    in_specs=[
        pl.BlockSpec((NC, BN, D), lambda i: (0, i, 0)),
        pl.BlockSpec((BN, D), lambda i: (i, 0)),
        pl.BlockSpec((BN, NW), lambda i: (i, 0)),
        pl.BlockSpec((1, D), lambda i: (0, 0)),
        pl.BlockSpec((1, 1, BN), lambda i: (i, 0, 0)),
    ],
    out_specs=[
        pl.BlockSpec((BN, D), lambda i: (i, 0)),
        pl.BlockSpec((G, D), lambda i: (0, 0)),
    ],
    out_shape=[
        jax.ShapeDtypeStruct((N, D), jnp.float32),
        jax.ShapeDtypeStruct((G, D), jnp.float32),
    ],
    scratch_shapes=[
        pltpu.VMEM((G, D), jnp.float32),
        pltpu.VMEM((G, D), jnp.float32),
    ],
)


def kernel(x, edge_index, batch, W0, b0, W1, b1):
    ei5 = edge_index.reshape(2, NW, NSTG, PCH, CH)
    zero_rows = jnp.zeros((RCH, D), jnp.float32)

    h0 = _tc0(x, W0)
    degp = _deg(ei5)
    deg_t = degp.T

    hs0 = _tc1(deg_t, h0)
    acc0 = _agg(hs0, ei5, zero_rows)
    hs1 = _tc2(acc0, hs0, deg_t, b0.reshape(1, D), W1)
    acc1 = _agg(hs1, ei5, zero_rows)
    h2, pooled = _tc3(acc1, hs1, deg_t, b1.reshape(1, D),
                      batch.reshape(NB, 1, BN))
    return (pooled, h2)


# speedup vs baseline: 1.3431x; 1.0279x over previous
import functools

import jax
import jax.numpy as jnp
from jax import lax
from jax.experimental import pallas as pl
from jax.experimental.pallas import tpu as pltpu
from jax.experimental.pallas import tpu_sc as plsc

N = 10000
E = 320000
D = 128
G = 64

NC = 2
NS = 16
NW = NC * NS
EPT = E // NW
CH = 80
NCH = EPT // CH
NSTG = 5
PCH = NCH // NSTG
NBUF = 3
RCH = 80
NRCH = N // RCH
DCH = 128
NDCH = 79
NPAD = NDCH * DCH

BN = 2000
NB = N // BN

_mesh = plsc.VectorSubcoreMesh(core_axis_name="c", subcore_axis_name="s")
_sc_params = pltpu.CompilerParams(needs_layout_passes=False)


def _deg_body(ei5, out_hbm, didx5, deg_v):
    c = lax.axis_index("c")
    s = lax.axis_index("s")
    wid = c * NS + s

    for p in range(NSTG):
        pltpu.sync_copy(ei5.at[1, wid, p], didx5.at[pl.ds(p * PCH, PCH)])

    zeros16 = jnp.zeros((16,), jnp.float32)

    def _zero(i, carry):
        deg_v[pl.ds(i * 16, 16)] = zeros16
        return carry

    lax.fori_loop(0, N // 16, _zero, 0)

    ones16 = jnp.ones((16,), jnp.float32)
    iota16 = lax.broadcasted_iota(jnp.int32, (16,), 0)

    def _count(t, carry):
        e = t * 16 + iota16
        idx = plsc.load_gather(didx5, [e // CH, e % CH])
        plsc.addupdate_scatter(deg_v, [idx], ones16)
        return carry

    lax.fori_loop(0, EPT // 16, _count, 0)

    pltpu.sync_copy(deg_v, out_hbm.at[wid])


_deg = functools.partial(
    pl.kernel,
    out_type=jax.ShapeDtypeStruct((NW, N), jnp.float32),
    mesh=_mesh,
    compiler_params=_sc_params,
    scratch_types=[
        pltpu.VMEM((NCH, CH), jnp.int32),
        pltpu.VMEM((N,), jnp.float32),
    ],
)(_deg_body)


def _agg_body(hs_hbm, ei5, zero_hbm, out_hbm, sidx, didx, rowsbuf, acc,
              semg0, semg1, semg2, sems0, sems1, sems2, semi):
    c = lax.axis_index("c")
    s = lax.axis_index("s")
    wid = c * NS + s
    rows = tuple(rowsbuf.at[pl.ds(b * CH, CH)] for b in range(NBUF))
    semg = (semg0, semg1, semg2)
    semsc = (sems0, sems1, sems2)
    rows0 = rows[0]

    pltpu.async_copy(ei5.at[0, wid, 0], sidx.at[0], semi)
    pltpu.async_copy(ei5.at[1, wid, 0], didx.at[0], semi)

    pltpu.sync_copy(zero_hbm, rows0.at[pl.ds(0, RCH)])
    nk = (NRCH - s + NS - 1) // NS

    def _zacc(k, carry):
        off = (s + k * NS) * RCH
        pltpu.sync_copy(rows0.at[pl.ds(0, RCH)], acc.at[pl.ds(off, RCH)])
        return carry

    lax.fori_loop(0, nk, _zacc, 0)
    pltpu.make_async_copy(ei5.at[0, wid, 0], sidx.at[0], semi).wait()
    pltpu.make_async_copy(ei5.at[1, wid, 0], didx.at[0], semi).wait()
    plsc.subcore_barrier()

    for p in range(NSTG):
        pb = p % 2
        sidx_p = sidx.at[pb]
        didx_p = didx.at[pb]
        if p < NSTG - 1:
            pltpu.async_copy(ei5.at[0, wid, p + 1], sidx.at[1 - pb], semi)
            pltpu.async_copy(ei5.at[1, wid, p + 1], didx.at[1 - pb], semi)
        for b in range(2):
            pltpu.async_copy(hs_hbm.at[sidx_p.at[b]], rows[b], semg[b])

        def _step(j, carry2, sidx_p=sidx_p, didx_p=didx_p):
            for u in range(NBUF):
                i = j * NBUF + u
                pltpu.make_async_copy(hs_hbm.at[sidx_p.at[i]], rows[u],
                                      semg[u]).wait()
                pltpu.async_copy(rows[u], acc.at[didx_p.at[i]], semsc[u],
                                 add=True)
                b2 = (u + 2) % NBUF

                @pl.when(i + 2 < PCH)
                def _():
                    @pl.when(i >= 1)
                    def _():
                        pltpu.make_async_copy(rows[b2], acc.at[didx_p.at[i - 1]],
                                              semsc[b2]).wait()

                    pltpu.async_copy(hs_hbm.at[sidx_p.at[i + 2]], rows[b2],
                                     semg[b2])

            return carry2

        lax.fori_loop(0, PCH // NBUF, _step, 0)
        last = PCH - 1
        lb = last % NBUF
        pltpu.make_async_copy(hs_hbm.at[sidx_p.at[last]], rows[lb],
                              semg[lb]).wait()
        pltpu.async_copy(rows[lb], acc.at[didx_p.at[last]], semsc[lb],
                         add=True)
        for i in range(PCH - NBUF, PCH):
            pltpu.make_async_copy(rows[i % NBUF], acc.at[didx_p.at[i]],
                                  semsc[i % NBUF]).wait()
        if p < NSTG - 1:
            pltpu.make_async_copy(ei5.at[0, wid, p + 1], sidx.at[1 - pb],
                                  semi).wait()
            pltpu.make_async_copy(ei5.at[1, wid, p + 1], didx.at[1 - pb],
                                  semi).wait()
    plsc.subcore_barrier()

    def _wb(k, carry):
        off = (s + k * NS) * RCH
        pltpu.sync_copy(acc.at[pl.ds(off, RCH)], out_hbm.at[c, pl.ds(off, RCH)])
        return carry

    lax.fori_loop(0, nk, _wb, 0)


_agg = functools.partial(
    pl.kernel,
    out_type=jax.ShapeDtypeStruct((NC, N, D), jnp.float32),
    mesh=_mesh,
    compiler_params=_sc_params,
    scratch_types=[
        pltpu.VMEM((2, PCH, CH), jnp.int32),
        pltpu.VMEM((2, PCH, CH), jnp.int32),
        pltpu.VMEM((NBUF * CH, D), jnp.float32),
        pltpu.VMEM_SHARED((N, D), jnp.float32),
        pltpu.SemaphoreType.DMA,
        pltpu.SemaphoreType.DMA,
        pltpu.SemaphoreType.DMA,
        pltpu.SemaphoreType.DMA,
        pltpu.SemaphoreType.DMA,
        pltpu.SemaphoreType.DMA,
        pltpu.SemaphoreType.DMA,
    ],
)(_agg_body)


def _dis_block(deg_ref):
    deg = jnp.sum(deg_ref[...], axis=1, keepdims=True) + 1.0
    return lax.rsqrt(deg)


def _tc0_body(x_ref, w_ref, h_ref):
    h_ref[...] = jnp.dot(x_ref[...], w_ref[...],
                         precision=lax.Precision.HIGHEST,
                         preferred_element_type=jnp.float32)


_tc0 = pl.pallas_call(
    _tc0_body,
    grid=(NB,),
    in_specs=[
        pl.BlockSpec((BN, D), lambda i: (i, 0)),
        pl.BlockSpec((D, D), lambda i: (0, 0)),
    ],
    out_specs=pl.BlockSpec((BN, D), lambda i: (i, 0)),
    out_shape=jax.ShapeDtypeStruct((N, D), jnp.float32),
)


def _tc1_body(deg_ref, h_ref, hs_ref):
    hs_ref[...] = h_ref[...] * _dis_block(deg_ref)


_tc1 = pl.pallas_call(
    _tc1_body,
    grid=(NB,),
    in_specs=[
        pl.BlockSpec((BN, NW), lambda i: (i, 0)),
        pl.BlockSpec((BN, D), lambda i: (i, 0)),
    ],
    out_specs=pl.BlockSpec((BN, D), lambda i: (i, 0)),
    out_shape=jax.ShapeDtypeStruct((N, D), jnp.float32),
)


def _tc2_body(acc_ref, hs_ref, deg_ref, b_ref, w_ref, out_ref):
    dis = _dis_block(deg_ref)
    t = (acc_ref[0] + acc_ref[1] + hs_ref[...]) * dis + b_ref[...]
    h1 = jnp.maximum(t, 0.0)
    out_ref[...] = jnp.dot(h1, w_ref[...], precision=lax.Precision.HIGHEST,
                           preferred_element_type=jnp.float32) * dis


_tc2 = pl.pallas_call(
    _tc2_body,
    grid=(NB,),
    in_specs=[
        pl.BlockSpec((NC, BN, D), lambda i: (0, i, 0)),
        pl.BlockSpec((BN, D), lambda i: (i, 0)),
        pl.BlockSpec((BN, NW), lambda i: (i, 0)),
        pl.BlockSpec((1, D), lambda i: (0, 0)),
        pl.BlockSpec((D, D), lambda i: (0, 0)),
    ],
    out_specs=pl.BlockSpec((BN, D), lambda i: (i, 0)),
    out_shape=jax.ShapeDtypeStruct((N, D), jnp.float32),
)


def _tc3_body(acc_ref, hs_ref, deg_ref, b_ref, batch_ref, h2_ref, pooled_ref,
              s_acc, c_acc):
    i = pl.program_id(0)
    dis = _dis_block(deg_ref)
    t = (acc_ref[0] + acc_ref[1] + hs_ref[...]) * dis + b_ref[...]
    h2 = jnp.maximum(t, 0.0)
    h2_ref[...] = h2

    bt = batch_ref[...].reshape(1, BN)
    gid = lax.broadcasted_iota(jnp.int32, (G, BN), 0)
    oh = jnp.where(gid == bt, 1.0, 0.0).astype(jnp.float32)
    sblk = jnp.dot(oh, h2, precision=lax.Precision.HIGHEST,
                   preferred_element_type=jnp.float32)
    cblk = jnp.broadcast_to(jnp.sum(oh, axis=1, keepdims=True), (G, D))

    @pl.when(i == 0)
    def _():
        s_acc[...] = sblk
        c_acc[...] = cblk

    @pl.when(i > 0)
    def _():
        s_acc[...] += sblk
        c_acc[...] += cblk

    @pl.when(i == NB - 1)
    def _():
        pooled_ref[...] = s_acc[...] / jnp.maximum(c_acc[...], 1.0)


_tc3 = pl.pallas_call(
    _tc3_body,
    grid=(NB,),
    in_specs=[
        pl.BlockSpec((NC, BN, D), lambda i: (0, i, 0)),
        pl.BlockSpec((BN, D), lambda i: (i, 0)),
        pl.BlockSpec((BN, NW), lambda i: (i, 0)),
        pl.BlockSpec((1, D), lambda i: (0, 0)),
        pl.BlockSpec((1, 1, BN), lambda i: (i, 0, 0)),
    ],
    out_specs=[
        pl.BlockSpec((BN, D), lambda i: (i, 0)),
        pl.BlockSpec((G, D), lambda i: (0, 0)),
    ],
    out_shape=[
        jax.ShapeDtypeStruct((N, D), jnp.float32),
        jax.ShapeDtypeStruct((G, D), jnp.float32),
    ],
    scratch_shapes=[
        pltpu.VMEM((G, D), jnp.float32),
        pltpu.VMEM((G, D), jnp.float32),
    ],
)


def kernel(x, edge_index, batch, W0, b0, W1, b1):
    ei5 = edge_index.reshape(2, NW, NSTG, PCH, CH)
    zero_rows = jnp.zeros((RCH, D), jnp.float32)

    h0 = _tc0(x, W0)
    degp = _deg(ei5)
    deg_t = degp.T

    hs0 = _tc1(deg_t, h0)
    acc0 = _agg(hs0, ei5, zero_rows)
    hs1 = _tc2(acc0, hs0, deg_t, b0.reshape(1, D), W1)
    acc1 = _agg(hs1, ei5, zero_rows)
    h2, pooled = _tc3(acc1, hs1, deg_t, b1.reshape(1, D),
                      batch.reshape(NB, 1, BN))
    return (pooled, h2)
